# baseline (speedup 1.0000x reference)
import functools

import jax
import jax.numpy as jnp
from jax import lax
from jax.experimental import pallas as pl
from jax.experimental.pallas import tpu as pltpu
from jax.experimental.pallas import tpu_sc as plsc

T = 2048
D = 1024
E = 8
K = 2
DFF = 2048
CAP = 640
NSLOT = E * CAP
NPAD = NSLOT + 8
DUMP = NSLOT

NC = 2
NS = 16
NW = NC * NS
TPW = T // NW
CCHUNK = 16
LANES = 16


def _router_body(x_ref, wg_ref, f0_ref, f1_ref, g0_ref, g1_ref):
    x = x_ref[...]
    logits = jnp.dot(x, wg_ref[...], preferred_element_type=jnp.float32)
    li = lax.broadcasted_iota(jnp.int32, (T, E), 1)
    m = jnp.max(logits, axis=1, keepdims=True)
    e = jnp.exp(logits - m)
    s = jnp.sum(e, axis=1, keepdims=True)
    v0 = jnp.max(e, axis=1, keepdims=True)
    i0 = jnp.min(jnp.where(e == v0, li, 127), axis=1, keepdims=True)
    e2 = jnp.where(li == i0, jnp.float32(-1.0), e)
    v1 = jnp.max(e2, axis=1, keepdims=True)
    i1 = jnp.min(jnp.where(e2 == v1, li, 127), axis=1, keepdims=True)
    tv0 = v0 / s
    tv1 = v1 / s
    den = tv0 + tv1 + jnp.float32(1e-9)
    g0 = tv0 / den
    g1 = tv1 / den
    li128 = lax.broadcasted_iota(jnp.int32, (T, 128), 1)
    oh0 = (li128 == i0).astype(jnp.float32)
    oh1 = (li128 == i1 + 64).astype(jnp.float32)
    ohc = oh0 + oh1
    z = jnp.concatenate([jnp.zeros((1, 128), jnp.float32), ohc[:-1]], axis=0)
    sh = 1
    while sh < T:
        z = z + jnp.concatenate(
            [jnp.zeros((sh, 128), jnp.float32), z[:-sh]], axis=0)
        sh *= 2
    tot0 = jnp.sum(oh0, axis=0, keepdims=True)
    loc0 = jnp.sum(z * oh0, axis=1, keepdims=True)
    loc1 = (jnp.sum(z * oh1, axis=1, keepdims=True)
            + jnp.sum(tot0 * (li128 == i1).astype(jnp.float32),
                      axis=1, keepdims=True))
    loc0 = loc0.astype(jnp.int32)
    loc1 = loc1.astype(jnp.int32)
    keep0 = loc0 < CAP
    keep1 = loc1 < CAP
    flat0 = i0 * CAP + jnp.minimum(loc0, CAP - 1)
    flat1 = i1 * CAP + jnp.minimum(loc1, CAP - 1)
    f0_ref[...] = jnp.where(keep0, flat0, DUMP)
    f1_ref[...] = jnp.where(keep1, flat1, DUMP)
    g0_ref[...] = jnp.broadcast_to(jnp.where(keep0, g0, 0.0), (T, LANES))
    g1_ref[...] = jnp.broadcast_to(jnp.where(keep1, g1, 0.0), (T, LANES))


def _router(x, wg):
    f0, f1, g0, g1 = pl.pallas_call(
        _router_body,
        out_shape=[
            jax.ShapeDtypeStruct((T, 1), jnp.int32),
            jax.ShapeDtypeStruct((T, 1), jnp.int32),
            jax.ShapeDtypeStruct((T, LANES), jnp.float32),
            jax.ShapeDtypeStruct((T, LANES), jnp.float32),
        ],
    )(x, wg)
    return f0.reshape(T), f1.reshape(T), g0, g1


def _dispatch_body(x_hbm, f0_hbm, f1_hbm, buf_hbm, rows_v, idx0_v, idx1_v,
                   sem):
    wid = lax.axis_index("c") * NS + lax.axis_index("s")
    base = wid * TPW
    pltpu.sync_copy(x_hbm.at[pl.ds(base, TPW)], rows_v)
    pltpu.sync_copy(f0_hbm.at[pl.ds(base, TPW)], idx0_v)
    pltpu.sync_copy(f1_hbm.at[pl.ds(base, TPW)], idx1_v)
    pltpu.async_copy(rows_v, buf_hbm.at[idx0_v], sem).wait()
    pltpu.async_copy(rows_v, buf_hbm.at[idx1_v], sem).wait()


def _dispatch(x, f0, f1):
    mesh = plsc.VectorSubcoreMesh(core_axis_name="c", subcore_axis_name="s")
    return pl.kernel(
        _dispatch_body,
        out_type=jax.ShapeDtypeStruct((NPAD, D), jnp.float32),
        mesh=mesh,
        scratch_types=[
            pltpu.VMEM((TPW, D), jnp.float32),
            pltpu.VMEM((TPW,), jnp.int32),
            pltpu.VMEM((TPW,), jnp.int32),
            pltpu.SemaphoreType.DMA,
        ],
    )(x, f0, f1)


JD = 1
DFJ = DFF // JD


def _ffn_body(buf_ref, w1_ref, b1_ref, w2_ref, b2_ref, y_ref):
    j = pl.program_id(1)
    h = jnp.dot(buf_ref[...], w1_ref[0], preferred_element_type=jnp.float32)
    h = jnp.maximum(h + b1_ref[0], 0.0)
    part = jnp.dot(h, w2_ref[0], preferred_element_type=jnp.float32)

    @pl.when(j == 0)
    def _():
        y_ref[...] = part + b2_ref[0]

    @pl.when(j != 0)
    def _():
        y_ref[...] += part


def _ffn(buf, fc1_w, fc1_b, fc2_w, fc2_b):
    return pl.pallas_call(
        _ffn_body,
        grid=(E, JD),
        in_specs=[
            pl.BlockSpec((CAP, D), lambda e, j: (e, 0)),
            pl.BlockSpec((1, D, DFJ), lambda e, j: (e, 0, j)),
            pl.BlockSpec((1, 1, DFJ), lambda e, j: (e, 0, j)),
            pl.BlockSpec((1, DFJ, D), lambda e, j: (e, j, 0)),
            pl.BlockSpec((1, 1, D), lambda e, j: (e, 0, 0)),
        ],
        out_specs=pl.BlockSpec((CAP, D), lambda e, j: (e, 0)),
        out_shape=jax.ShapeDtypeStruct((NPAD, D), jnp.float32),
        compiler_params=pltpu.CompilerParams(
            dimension_semantics=("arbitrary", "arbitrary")),
    )(buf, fc1_w, fc1_b.reshape(E, 1, DFF), fc2_w, fc2_b.reshape(E, 1, D))


NCH = TPW // CCHUNK


def _combine_body(y_hbm, f0_hbm, f1_hbm, g0_hbm, g1_hbm, out_hbm,
                  idx0_v, idx1_v, g0_v, g1_v,
                  rows0_a, rows1_a, rows0_b, rows1_b,
                  sem0a, sem1a, sem0b, sem1b, semw_a, semw_b):
    wid = lax.axis_index("c") * NS + lax.axis_index("s")
    base = wid * TPW
    pltpu.sync_copy(f0_hbm.at[pl.ds(base, TPW)], idx0_v)
    pltpu.sync_copy(f1_hbm.at[pl.ds(base, TPW)], idx1_v)
    pltpu.sync_copy(g0_hbm.at[pl.ds(base, TPW)], g0_v)
    pltpu.sync_copy(g1_hbm.at[pl.ds(base, TPW)], g1_v)

    bufs = [(rows0_a, rows1_a, sem0a, sem1a), (rows0_b, rows1_b, sem0b, sem1b)]
    wsems = [semw_a, semw_b]

    def gather(c, r0, r1, s0, s1):
        t0 = c * CCHUNK
        h0 = pltpu.async_copy(y_hbm.at[idx0_v.at[pl.ds(t0, CCHUNK)]], r0, s0)
        h1 = pltpu.async_copy(y_hbm.at[idx1_v.at[pl.ds(t0, CCHUNK)]], r1, s1)
        return h0, h1

    handles = [None, None]
    wh = [None, None]
    handles[0] = gather(0, *bufs[0])
    for c in range(NCH):
        cur = c % 2
        nxt = (c + 1) % 2
        if c + 1 < NCH:
            if wh[nxt] is not None:
                wh[nxt].wait()
            handles[nxt] = gather(c + 1, *bufs[nxt])
        r0, r1, _, _ = bufs[cur]
        handles[cur][0].wait()
        handles[cur][1].wait()

        def token(i, _):
            t = c * CCHUNK + i
            g0b = g0_v[t, :]
            g1b = g1_v[t, :]
            m0 = g0b > 0.0
            m1 = g1b > 0.0
            for j in range(D // LANES):
                sl = pl.ds(j * LANES, LANES)
                r0[i, sl] = (jnp.where(m0, g0b * r0[i, sl], 0.0)
                             + jnp.where(m1, g1b * r1[i, sl], 0.0))
            return 0

        lax.fori_loop(0, CCHUNK, token, 0)
        wh[cur] = pltpu.async_copy(
            r0, out_hbm.at[pl.ds(base + c * CCHUNK, CCHUNK)], wsems[cur])
    for h in wh:
        if h is not None:
            h.wait()


def _combine(y, f0, f1, g0, g1):
    mesh = plsc.VectorSubcoreMesh(core_axis_name="c", subcore_axis_name="s")
    return pl.kernel(
        _combine_body,
        out_type=jax.ShapeDtypeStruct((T, D), jnp.float32),
        mesh=mesh,
        scratch_types=[
            pltpu.VMEM((TPW,), jnp.int32),
            pltpu.VMEM((TPW,), jnp.int32),
            pltpu.VMEM((TPW, LANES), jnp.float32),
            pltpu.VMEM((TPW, LANES), jnp.float32),
            pltpu.VMEM((CCHUNK, D), jnp.float32),
            pltpu.VMEM((CCHUNK, D), jnp.float32),
            pltpu.VMEM((CCHUNK, D), jnp.float32),
            pltpu.VMEM((CCHUNK, D), jnp.float32),
            pltpu.SemaphoreType.DMA,
            pltpu.SemaphoreType.DMA,
            pltpu.SemaphoreType.DMA,
            pltpu.SemaphoreType.DMA,
            pltpu.SemaphoreType.DMA,
            pltpu.SemaphoreType.DMA,
        ],
    )(y, f0, f1, g0, g1)


@jax.jit
def kernel(x, wg, fc1_w, fc1_b, fc2_w, fc2_b):
    f0, f1, g0, g1 = _router(x, wg)
    buf = _dispatch(x, f0, f1)
    y = _ffn(buf, fc1_w, fc1_b, fc2_w, fc2_b)
    return _combine(y, f0, f1, g0, g1)


# speedup vs baseline: 1.0262x; 1.0262x over previous
import functools

import jax
import jax.numpy as jnp
from jax import lax
from jax.experimental import pallas as pl
from jax.experimental.pallas import tpu as pltpu
from jax.experimental.pallas import tpu_sc as plsc

T = 2048
D = 1024
E = 8
K = 2
DFF = 2048
CAP = 640
NSLOT = E * CAP
NPAD = NSLOT + 8
DUMP = NSLOT

NC = 2
NS = 16
NW = NC * NS
TPW = T // NW
CCHUNK = 16
LANES = 16


def _router_body(x_ref, wg_ref, f0_ref, f1_ref, g0_ref, g1_ref):
    x = x_ref[...]
    logits = jnp.dot(x, wg_ref[...], preferred_element_type=jnp.float32)
    li = lax.broadcasted_iota(jnp.int32, (T, E), 1)
    m = jnp.max(logits, axis=1, keepdims=True)
    e = jnp.exp(logits - m)
    s = jnp.sum(e, axis=1, keepdims=True)
    v0 = jnp.max(e, axis=1, keepdims=True)
    i0 = jnp.min(jnp.where(e == v0, li, 127), axis=1, keepdims=True)
    e2 = jnp.where(li == i0, jnp.float32(-1.0), e)
    v1 = jnp.max(e2, axis=1, keepdims=True)
    i1 = jnp.min(jnp.where(e2 == v1, li, 127), axis=1, keepdims=True)
    tv0 = v0 / s
    tv1 = v1 / s
    den = tv0 + tv1 + jnp.float32(1e-9)
    g0 = tv0 / den
    g1 = tv1 / den
    li128 = lax.broadcasted_iota(jnp.int32, (T, 128), 1)
    oh0 = (li128 == i0).astype(jnp.float32)
    oh1 = (li128 == i1 + 64).astype(jnp.float32)
    ohc = oh0 + oh1
    z = jnp.concatenate([jnp.zeros((1, 128), jnp.float32), ohc[:-1]], axis=0)
    sh = 1
    while sh < T:
        z = z + jnp.concatenate(
            [jnp.zeros((sh, 128), jnp.float32), z[:-sh]], axis=0)
        sh *= 2
    tot0 = jnp.sum(oh0, axis=0, keepdims=True)
    loc0 = jnp.sum(z * oh0, axis=1, keepdims=True)
    loc1 = (jnp.sum(z * oh1, axis=1, keepdims=True)
            + jnp.sum(tot0 * (li128 == i1).astype(jnp.float32),
                      axis=1, keepdims=True))
    loc0 = loc0.astype(jnp.int32)
    loc1 = loc1.astype(jnp.int32)
    keep0 = loc0 < CAP
    keep1 = loc1 < CAP
    flat0 = i0 * CAP + jnp.minimum(loc0, CAP - 1)
    flat1 = i1 * CAP + jnp.minimum(loc1, CAP - 1)
    f0_ref[...] = jnp.where(keep0, flat0, DUMP)
    f1_ref[...] = jnp.where(keep1, flat1, DUMP)
    g0_ref[...] = jnp.broadcast_to(jnp.where(keep0, g0, 0.0), (T, LANES))
    g1_ref[...] = jnp.broadcast_to(jnp.where(keep1, g1, 0.0), (T, LANES))


def _router(x, wg):
    f0, f1, g0, g1 = pl.pallas_call(
        _router_body,
        out_shape=[
            jax.ShapeDtypeStruct((T, 1), jnp.int32),
            jax.ShapeDtypeStruct((T, 1), jnp.int32),
            jax.ShapeDtypeStruct((T, LANES), jnp.float32),
            jax.ShapeDtypeStruct((T, LANES), jnp.float32),
        ],
    )(x, wg)
    return f0.reshape(T), f1.reshape(T), g0, g1


def _dispatch_body(x_hbm, f0_hbm, f1_hbm, buf_hbm, rows_v, idx0_v, idx1_v,
                   semr, sem0, sem1):
    wid = lax.axis_index("c") * NS + lax.axis_index("s")
    base = wid * TPW
    hr = pltpu.async_copy(x_hbm.at[pl.ds(base, TPW)], rows_v, semr)
    pltpu.sync_copy(f0_hbm.at[pl.ds(base, TPW)], idx0_v)
    pltpu.sync_copy(f1_hbm.at[pl.ds(base, TPW)], idx1_v)
    hr.wait()
    h0 = pltpu.async_copy(rows_v, buf_hbm.at[idx0_v], sem0)
    h1 = pltpu.async_copy(rows_v, buf_hbm.at[idx1_v], sem1)
    h0.wait()
    h1.wait()


def _dispatch(x, f0, f1):
    mesh = plsc.VectorSubcoreMesh(core_axis_name="c", subcore_axis_name="s")
    return pl.kernel(
        _dispatch_body,
        out_type=jax.ShapeDtypeStruct((NPAD, D), jnp.float32),
        mesh=mesh,
        scratch_types=[
            pltpu.VMEM((TPW, D), jnp.float32),
            pltpu.VMEM((TPW,), jnp.int32),
            pltpu.VMEM((TPW,), jnp.int32),
            pltpu.SemaphoreType.DMA,
            pltpu.SemaphoreType.DMA,
            pltpu.SemaphoreType.DMA,
        ],
    )(x, f0, f1)


JD = 1
DFJ = DFF // JD


def _ffn_body(buf_ref, w1_ref, b1_ref, w2_ref, b2_ref, y_ref):
    j = pl.program_id(1)
    h = jnp.dot(buf_ref[...], w1_ref[0], preferred_element_type=jnp.float32)
    h = jnp.maximum(h + b1_ref[0], 0.0)
    part = jnp.dot(h, w2_ref[0], preferred_element_type=jnp.float32)

    @pl.when(j == 0)
    def _():
        y_ref[...] = part + b2_ref[0]

    @pl.when(j != 0)
    def _():
        y_ref[...] += part


def _ffn(buf, fc1_w, fc1_b, fc2_w, fc2_b):
    return pl.pallas_call(
        _ffn_body,
        grid=(E, JD),
        in_specs=[
            pl.BlockSpec((CAP, D), lambda e, j: (e, 0)),
            pl.BlockSpec((1, D, DFJ), lambda e, j: (e, 0, j)),
            pl.BlockSpec((1, 1, DFJ), lambda e, j: (e, 0, j)),
            pl.BlockSpec((1, DFJ, D), lambda e, j: (e, j, 0)),
            pl.BlockSpec((1, 1, D), lambda e, j: (e, 0, 0)),
        ],
        out_specs=pl.BlockSpec((CAP, D), lambda e, j: (e, 0)),
        out_shape=jax.ShapeDtypeStruct((NPAD, D), jnp.float32),
        compiler_params=pltpu.CompilerParams(
            dimension_semantics=("arbitrary", "arbitrary")),
    )(buf, fc1_w, fc1_b.reshape(E, 1, DFF), fc2_w, fc2_b.reshape(E, 1, D))


NCH = TPW // CCHUNK


def _combine_body(y_hbm, f0_hbm, f1_hbm, g0_hbm, g1_hbm, out_hbm,
                  idx0_v, idx1_v, g0_v, g1_v,
                  rows0_a, rows1_a, rows0_b, rows1_b,
                  sem0a, sem1a, sem0b, sem1b, semw_a, semw_b):
    wid = lax.axis_index("c") * NS + lax.axis_index("s")
    base = wid * TPW
    pltpu.sync_copy(f0_hbm.at[pl.ds(base, TPW)], idx0_v)
    pltpu.sync_copy(f1_hbm.at[pl.ds(base, TPW)], idx1_v)
    pltpu.sync_copy(g0_hbm.at[pl.ds(base, TPW)], g0_v)
    pltpu.sync_copy(g1_hbm.at[pl.ds(base, TPW)], g1_v)

    bufs = [(rows0_a, rows1_a, sem0a, sem1a), (rows0_b, rows1_b, sem0b, sem1b)]
    wsems = [semw_a, semw_b]

    def gather(c, r0, r1, s0, s1):
        t0 = c * CCHUNK
        h0 = pltpu.async_copy(y_hbm.at[idx0_v.at[pl.ds(t0, CCHUNK)]], r0, s0)
        h1 = pltpu.async_copy(y_hbm.at[idx1_v.at[pl.ds(t0, CCHUNK)]], r1, s1)
        return h0, h1

    handles = [None, None]
    wh = [None, None]
    handles[0] = gather(0, *bufs[0])
    for c in range(NCH):
        cur = c % 2
        nxt = (c + 1) % 2
        if c + 1 < NCH:
            if wh[nxt] is not None:
                wh[nxt].wait()
            handles[nxt] = gather(c + 1, *bufs[nxt])
        r0, r1, _, _ = bufs[cur]
        handles[cur][0].wait()
        handles[cur][1].wait()

        def token(i, _):
            t = c * CCHUNK + i
            g0b = g0_v[t, :]
            g1b = g1_v[t, :]
            m0 = g0b > 0.0
            m1 = g1b > 0.0

            def chunk16(j):
                sl = pl.ds(j * LANES, LANES)
                r0[i, sl] = (jnp.where(m0, g0b * r0[i, sl], 0.0)
                             + jnp.where(m1, g1b * r1[i, sl], 0.0))

            plsc.parallel_loop(0, D // LANES, 1, unroll=8)(chunk16)
            return 0

        lax.fori_loop(0, CCHUNK, token, 0)
        wh[cur] = pltpu.async_copy(
            r0, out_hbm.at[pl.ds(base + c * CCHUNK, CCHUNK)], wsems[cur])
    for h in wh:
        if h is not None:
            h.wait()


def _combine(y, f0, f1, g0, g1):
    mesh = plsc.VectorSubcoreMesh(core_axis_name="c", subcore_axis_name="s")
    return pl.kernel(
        _combine_body,
        out_type=jax.ShapeDtypeStruct((T, D), jnp.float32),
        mesh=mesh,
        scratch_types=[
            pltpu.VMEM((TPW,), jnp.int32),
            pltpu.VMEM((TPW,), jnp.int32),
            pltpu.VMEM((TPW, LANES), jnp.float32),
            pltpu.VMEM((TPW, LANES), jnp.float32),
            pltpu.VMEM((CCHUNK, D), jnp.float32),
            pltpu.VMEM((CCHUNK, D), jnp.float32),
            pltpu.VMEM((CCHUNK, D), jnp.float32),
            pltpu.VMEM((CCHUNK, D), jnp.float32),
            pltpu.SemaphoreType.DMA,
            pltpu.SemaphoreType.DMA,
            pltpu.SemaphoreType.DMA,
            pltpu.SemaphoreType.DMA,
            pltpu.SemaphoreType.DMA,
            pltpu.SemaphoreType.DMA,
        ],
    )(y, f0, f1, g0, g1)


@jax.jit
def kernel(x, wg, fc1_w, fc1_b, fc2_w, fc2_b):
    f0, f1, g0, g1 = _router(x, wg)
    buf = _dispatch(x, f0, f1)
    y = _ffn(buf, fc1_w, fc1_b, fc2_w, fc2_b)
    return _combine(y, f0, f1, g0, g1)


# speedup vs baseline: 1.0487x; 1.0220x over previous
import functools

import jax
import jax.numpy as jnp
from jax import lax
from jax.experimental import pallas as pl
from jax.experimental.pallas import tpu as pltpu
from jax.experimental.pallas import tpu_sc as plsc

T = 2048
D = 1024
DH = D // 2
E = 8
K = 2
DFF = 2048
CAP = 640
NSLOT = E * CAP
NPAD = NSLOT + 8
DUMP = NSLOT

NC = 2
NS = 16
NW = NC * NS
TPW = T // NW
CCHUNK = 32
LANES = 16
BLANES = 32


def _rne16(xi):
    return xi + 0x7FFF + ((xi >> 16) & 1)


def _pack_pairs(xf):
    xi = lax.bitcast_convert_type(xf, jnp.int32)
    b = lax.shift_right_logical(_rne16(xi), 16)
    lo = b[:, :DH]
    hi = b[:, DH:]
    return jnp.bitwise_or(lax.shift_left(hi, 16), lo)


def _unpack_pairs(xi):
    lo = lax.bitcast_convert_type(lax.shift_left(xi, 16), jnp.float32)
    hi = lax.bitcast_convert_type(
        jnp.bitwise_and(xi, jnp.int32(-65536)), jnp.float32)
    return jnp.concatenate([lo, hi], axis=1)


def _router_body(x_ref, wg_ref, f0_ref, f1_ref, g0_ref, g1_ref, xb_ref):
    x = x_ref[...]
    xb_ref[...] = _pack_pairs(x)
    logits = jnp.dot(x, wg_ref[...], preferred_element_type=jnp.float32)
    li = lax.broadcasted_iota(jnp.int32, (T, E), 1)
    m = jnp.max(logits, axis=1, keepdims=True)
    e = jnp.exp(logits - m)
    s = jnp.sum(e, axis=1, keepdims=True)
    v0 = jnp.max(e, axis=1, keepdims=True)
    i0 = jnp.min(jnp.where(e == v0, li, 127), axis=1, keepdims=True)
    e2 = jnp.where(li == i0, jnp.float32(-1.0), e)
    v1 = jnp.max(e2, axis=1, keepdims=True)
    i1 = jnp.min(jnp.where(e2 == v1, li, 127), axis=1, keepdims=True)
    tv0 = v0 / s
    tv1 = v1 / s
    den = tv0 + tv1 + jnp.float32(1e-9)
    g0 = tv0 / den
    g1 = tv1 / den
    li128 = lax.broadcasted_iota(jnp.int32, (T, 128), 1)
    oh0 = (li128 == i0).astype(jnp.float32)
    oh1 = (li128 == i1 + 64).astype(jnp.float32)
    ohc = oh0 + oh1
    z = jnp.concatenate([jnp.zeros((1, 128), jnp.float32), ohc[:-1]], axis=0)
    sh = 1
    while sh < T:
        z = z + jnp.concatenate(
            [jnp.zeros((sh, 128), jnp.float32), z[:-sh]], axis=0)
        sh *= 2
    tot0 = jnp.sum(oh0, axis=0, keepdims=True)
    loc0 = jnp.sum(z * oh0, axis=1, keepdims=True)
    loc1 = (jnp.sum(z * oh1, axis=1, keepdims=True)
            + jnp.sum(tot0 * (li128 == i1).astype(jnp.float32),
                      axis=1, keepdims=True))
    loc0 = loc0.astype(jnp.int32)
    loc1 = loc1.astype(jnp.int32)
    keep0 = loc0 < CAP
    keep1 = loc1 < CAP
    flat0 = i0 * CAP + jnp.minimum(loc0, CAP - 1)
    flat1 = i1 * CAP + jnp.minimum(loc1, CAP - 1)
    f0_ref[...] = jnp.where(keep0, flat0, DUMP)
    f1_ref[...] = jnp.where(keep1, flat1, DUMP)
    g0_ref[...] = jnp.where(keep0, g0, 0.0)
    g1_ref[...] = jnp.where(keep1, g1, 0.0)


def _router(x, wg):
    f0, f1, g0, g1, xb = pl.pallas_call(
        _router_body,
        out_shape=[
            jax.ShapeDtypeStruct((T, 1), jnp.int32),
            jax.ShapeDtypeStruct((T, 1), jnp.int32),
            jax.ShapeDtypeStruct((T, 1), jnp.float32),
            jax.ShapeDtypeStruct((T, 1), jnp.float32),
            jax.ShapeDtypeStruct((T, DH), jnp.int32),
        ],
    )(x, wg)
    return f0.reshape(T), f1.reshape(T), g0, g1, xb


def _dispatch_body(x_hbm, f0_hbm, f1_hbm, buf_hbm, rows_v, idx0_v, idx1_v,
                   semr, sem0, sem1):
    wid = lax.axis_index("c") * NS + lax.axis_index("s")
    base = wid * TPW
    hr = pltpu.async_copy(x_hbm.at[pl.ds(base, TPW)], rows_v, semr)
    pltpu.sync_copy(f0_hbm.at[pl.ds(base, TPW)], idx0_v)
    pltpu.sync_copy(f1_hbm.at[pl.ds(base, TPW)], idx1_v)
    hr.wait()
    h0 = pltpu.async_copy(rows_v, buf_hbm.at[idx0_v], sem0)
    h1 = pltpu.async_copy(rows_v, buf_hbm.at[idx1_v], sem1)
    h0.wait()
    h1.wait()


def _dispatch(xb, f0, f1):
    mesh = plsc.VectorSubcoreMesh(core_axis_name="c", subcore_axis_name="s")
    return pl.kernel(
        _dispatch_body,
        out_type=jax.ShapeDtypeStruct((NPAD, DH), jnp.int32),
        mesh=mesh,
        scratch_types=[
            pltpu.VMEM((TPW, DH), jnp.int32),
            pltpu.VMEM((TPW,), jnp.int32),
            pltpu.VMEM((TPW,), jnp.int32),
            pltpu.SemaphoreType.DMA,
            pltpu.SemaphoreType.DMA,
            pltpu.SemaphoreType.DMA,
        ],
    )(xb, f0, f1)


def _ffn_body(buf_ref, w1_ref, b1_ref, w2_ref, b2_ref, y_ref):
    lhs = _unpack_pairs(buf_ref[...])
    h = jnp.dot(lhs, w1_ref[0], preferred_element_type=jnp.float32)
    h = jnp.maximum(h + b1_ref[0], 0.0)
    y = jnp.dot(h, w2_ref[0], preferred_element_type=jnp.float32) + b2_ref[0]
    y_ref[...] = _pack_pairs(y)


def _ffn(buf, fc1_w, fc1_b, fc2_w, fc2_b):
    return pl.pallas_call(
        _ffn_body,
        grid=(E,),
        in_specs=[
            pl.BlockSpec((CAP, DH), lambda e: (e, 0)),
            pl.BlockSpec((1, D, DFF), lambda e: (e, 0, 0)),
            pl.BlockSpec((1, 1, DFF), lambda e: (e, 0, 0)),
            pl.BlockSpec((1, DFF, D), lambda e: (e, 0, 0)),
            pl.BlockSpec((1, 1, D), lambda e: (e, 0, 0)),
        ],
        out_specs=pl.BlockSpec((CAP, DH), lambda e: (e, 0)),
        out_shape=jax.ShapeDtypeStruct((NPAD, DH), jnp.int32),
        compiler_params=pltpu.CompilerParams(
            dimension_semantics=("arbitrary",)),
    )(buf, fc1_w, fc1_b.reshape(E, 1, DFF), fc2_w, fc2_b.reshape(E, 1, D))


NCH = TPW // CCHUNK


def _gather_body(y_hbm, f0_hbm, f1_hbm, r0_hbm, r1_hbm,
                 idx0_v, idx1_v,
                 rows0_a, rows1_a, rows0_b, rows1_b,
                 sem0a, sem1a, sem0b, sem1b, semw_a, semw_b):
    wid = lax.axis_index("c") * NS + lax.axis_index("s")
    base = wid * TPW
    pltpu.sync_copy(f0_hbm.at[pl.ds(base, TPW)], idx0_v)
    pltpu.sync_copy(f1_hbm.at[pl.ds(base, TPW)], idx1_v)

    bufs = [(rows0_a, rows1_a, sem0a, sem1a), (rows0_b, rows1_b, sem0b, sem1b)]
    wsems = [semw_a, semw_b]

    def gather(c, r0, r1, s0, s1):
        t0 = c * CCHUNK
        h0 = pltpu.async_copy(y_hbm.at[idx0_v.at[pl.ds(t0, CCHUNK)]], r0, s0)
        h1 = pltpu.async_copy(y_hbm.at[idx1_v.at[pl.ds(t0, CCHUNK)]], r1, s1)
        return h0, h1

    handles = [None, None]
    wh = [None, None]
    handles[0] = gather(0, *bufs[0])
    for c in range(NCH):
        cur = c % 2
        nxt = (c + 1) % 2
        if c + 1 < NCH:
            if wh[nxt] is not None:
                wh[nxt][0].wait()
                wh[nxt][1].wait()
            handles[nxt] = gather(c + 1, *bufs[nxt])
        r0, r1, _, _ = bufs[cur]
        handles[cur][0].wait()
        handles[cur][1].wait()
        t0 = base + c * CCHUNK
        w0 = pltpu.async_copy(r0, r0_hbm.at[pl.ds(t0, CCHUNK)], wsems[cur])
        w1 = pltpu.async_copy(r1, r1_hbm.at[pl.ds(t0, CCHUNK)], wsems[cur])
        wh[cur] = (w0, w1)
    for h in wh:
        if h is not None:
            h[0].wait()
            h[1].wait()


def _gather(y, f0, f1):
    mesh = plsc.VectorSubcoreMesh(core_axis_name="c", subcore_axis_name="s")
    return pl.kernel(
        _gather_body,
        out_type=[
            jax.ShapeDtypeStruct((T, DH), jnp.int32),
            jax.ShapeDtypeStruct((T, DH), jnp.int32),
        ],
        mesh=mesh,
        scratch_types=[
            pltpu.VMEM((TPW,), jnp.int32),
            pltpu.VMEM((TPW,), jnp.int32),
            pltpu.VMEM((CCHUNK, DH), jnp.int32),
            pltpu.VMEM((CCHUNK, DH), jnp.int32),
            pltpu.VMEM((CCHUNK, DH), jnp.int32),
            pltpu.VMEM((CCHUNK, DH), jnp.int32),
            pltpu.SemaphoreType.DMA,
            pltpu.SemaphoreType.DMA,
            pltpu.SemaphoreType.DMA,
            pltpu.SemaphoreType.DMA,
            pltpu.SemaphoreType.DMA,
            pltpu.SemaphoreType.DMA,
        ],
    )(y, f0, f1)


MIXB = 256


def _mix_body(r0_ref, r1_ref, g0_ref, g1_ref, out_ref):
    g0 = g0_ref[...]
    g1 = g1_ref[...]
    r0 = _unpack_pairs(r0_ref[...])
    r1 = _unpack_pairs(r1_ref[...])
    out_ref[...] = (jnp.where(g0 > 0, g0 * r0, 0.0)
                    + jnp.where(g1 > 0, g1 * r1, 0.0))


def _mix(r0, r1, g0, g1):
    return pl.pallas_call(
        _mix_body,
        grid=(T // MIXB,),
        in_specs=[
            pl.BlockSpec((MIXB, DH), lambda i: (i, 0)),
            pl.BlockSpec((MIXB, DH), lambda i: (i, 0)),
            pl.BlockSpec((MIXB, 1), lambda i: (i, 0)),
            pl.BlockSpec((MIXB, 1), lambda i: (i, 0)),
        ],
        out_specs=pl.BlockSpec((MIXB, D), lambda i: (i, 0)),
        out_shape=jax.ShapeDtypeStruct((T, D), jnp.float32),
        compiler_params=pltpu.CompilerParams(
            dimension_semantics=("arbitrary",)),
    )(r0, r1, g0, g1)


@jax.jit
def kernel(x, wg, fc1_w, fc1_b, fc2_w, fc2_b):
    f0, f1, g0, g1, xb = _router(x, wg)
    buf = _dispatch(xb, f0, f1)
    y = _ffn(buf, fc1_w, fc1_b, fc2_w, fc2_b)
    r0, r1 = _gather(y, f0, f1)
    return _mix(r0, r1, g0, g1)


# speedup vs baseline: 1.0887x; 1.0381x over previous
import functools

import jax
import jax.numpy as jnp
from jax import lax
from jax.experimental import pallas as pl
from jax.experimental.pallas import tpu as pltpu
from jax.experimental.pallas import tpu_sc as plsc

T = 2048
D = 1024
DH = D // 2
E = 8
K = 2
DFF = 2048
CAP = 640
NSLOT = E * CAP
NPAD = NSLOT + 8
DUMP = NSLOT

NC = 2
NS = 16
NW = NC * NS
TPW = T // NW
CCHUNK = 32
LANES = 16
BLANES = 32


def _rne16(xi):
    return xi + 0x7FFF + ((xi >> 16) & 1)


def _pack_pairs(xf):
    xi = lax.bitcast_convert_type(xf, jnp.int32)
    b = lax.shift_right_logical(_rne16(xi), 16)
    lo = b[:, :DH]
    hi = b[:, DH:]
    return jnp.bitwise_or(lax.shift_left(hi, 16), lo)


def _unpack_pairs(xi):
    lo = lax.bitcast_convert_type(lax.shift_left(xi, 16), jnp.float32)
    hi = lax.bitcast_convert_type(
        jnp.bitwise_and(xi, jnp.int32(-65536)), jnp.float32)
    return jnp.concatenate([lo, hi], axis=1)


def _router_body(x_ref, wg_ref, f0_ref, f1_ref, g0_ref, g1_ref, xb_ref):
    x = x_ref[...]
    xb_ref[...] = _pack_pairs(x)
    logits = jnp.dot(x, wg_ref[...], preferred_element_type=jnp.float32)
    li = lax.broadcasted_iota(jnp.int32, (T, E), 1)
    m = jnp.max(logits, axis=1, keepdims=True)
    e = jnp.exp(logits - m)
    s = jnp.sum(e, axis=1, keepdims=True)
    v0 = jnp.max(e, axis=1, keepdims=True)
    i0 = jnp.min(jnp.where(e == v0, li, 127), axis=1, keepdims=True)
    e2 = jnp.where(li == i0, jnp.float32(-1.0), e)
    v1 = jnp.max(e2, axis=1, keepdims=True)
    i1 = jnp.min(jnp.where(e2 == v1, li, 127), axis=1, keepdims=True)
    tv0 = v0 / s
    tv1 = v1 / s
    den = tv0 + tv1 + jnp.float32(1e-9)
    g0 = tv0 / den
    g1 = tv1 / den
    li128 = lax.broadcasted_iota(jnp.int32, (T, 128), 1)
    oh0 = (li128 == i0).astype(jnp.float32)
    oh1 = (li128 == i1 + 64).astype(jnp.float32)
    ohc = oh0 + oh1
    z = jnp.concatenate([jnp.zeros((1, 128), jnp.float32), ohc[:-1]], axis=0)
    sh = 1
    while sh < T:
        z = z + jnp.concatenate(
            [jnp.zeros((sh, 128), jnp.float32), z[:-sh]], axis=0)
        sh *= 2
    tot0 = jnp.sum(oh0, axis=0, keepdims=True)
    loc0 = jnp.sum(z * oh0, axis=1, keepdims=True)
    loc1 = (jnp.sum(z * oh1, axis=1, keepdims=True)
            + jnp.sum(tot0 * (li128 == i1).astype(jnp.float32),
                      axis=1, keepdims=True))
    loc0 = loc0.astype(jnp.int32)
    loc1 = loc1.astype(jnp.int32)
    keep0 = loc0 < CAP
    keep1 = loc1 < CAP
    flat0 = i0 * CAP + jnp.minimum(loc0, CAP - 1)
    flat1 = i1 * CAP + jnp.minimum(loc1, CAP - 1)
    f0_ref[...] = jnp.where(keep0, flat0, DUMP).reshape(T)
    f1_ref[...] = jnp.where(keep1, flat1, DUMP).reshape(T)
    g0_ref[...] = jnp.where(keep0, g0, 0.0)
    g1_ref[...] = jnp.where(keep1, g1, 0.0)


def _router(x, wg):
    f0, f1, g0, g1, xb = pl.pallas_call(
        _router_body,
        out_shape=[
            jax.ShapeDtypeStruct((T,), jnp.int32),
            jax.ShapeDtypeStruct((T,), jnp.int32),
            jax.ShapeDtypeStruct((T, 1), jnp.float32),
            jax.ShapeDtypeStruct((T, 1), jnp.float32),
            jax.ShapeDtypeStruct((T, DH), jnp.int32),
        ],
    )(x, wg)
    return f0, f1, g0, g1, xb


def _dispatch_body(x_hbm, f0_hbm, f1_hbm, buf_hbm, rows_v, idx0_v, idx1_v,
                   semr, sem0, sem1):
    wid = lax.axis_index("c") * NS + lax.axis_index("s")
    base = wid * TPW
    hr = pltpu.async_copy(x_hbm.at[pl.ds(base, TPW)], rows_v, semr)
    pltpu.sync_copy(f0_hbm.at[pl.ds(base, TPW)], idx0_v)
    pltpu.sync_copy(f1_hbm.at[pl.ds(base, TPW)], idx1_v)
    hr.wait()
    h0 = pltpu.async_copy(rows_v, buf_hbm.at[idx0_v], sem0)
    h1 = pltpu.async_copy(rows_v, buf_hbm.at[idx1_v], sem1)
    h0.wait()
    h1.wait()


def _dispatch(xb, f0, f1):
    mesh = plsc.VectorSubcoreMesh(core_axis_name="c", subcore_axis_name="s")
    return pl.kernel(
        _dispatch_body,
        out_type=jax.ShapeDtypeStruct((NPAD, DH), jnp.int32),
        mesh=mesh,
        scratch_types=[
            pltpu.VMEM((TPW, DH), jnp.int32),
            pltpu.VMEM((TPW,), jnp.int32),
            pltpu.VMEM((TPW,), jnp.int32),
            pltpu.SemaphoreType.DMA,
            pltpu.SemaphoreType.DMA,
            pltpu.SemaphoreType.DMA,
        ],
    )(xb, f0, f1)


def _ffn_body(buf_ref, w1_ref, b1_ref, w2_ref, b2_ref, y_ref):
    lhs = _unpack_pairs(buf_ref[...])
    h = jnp.dot(lhs, w1_ref[0], preferred_element_type=jnp.float32)
    h = jnp.maximum(h + b1_ref[0], 0.0)
    y = jnp.dot(h, w2_ref[0], preferred_element_type=jnp.float32) + b2_ref[0]
    y_ref[...] = _pack_pairs(y)


def _ffn(buf, fc1_w, fc1_b, fc2_w, fc2_b):
    return pl.pallas_call(
        _ffn_body,
        grid=(E,),
        in_specs=[
            pl.BlockSpec((CAP, DH), lambda e: (e, 0)),
            pl.BlockSpec((1, D, DFF), lambda e: (e, 0, 0)),
            pl.BlockSpec((1, 1, DFF), lambda e: (e, 0, 0)),
            pl.BlockSpec((1, DFF, D), lambda e: (e, 0, 0)),
            pl.BlockSpec((1, 1, D), lambda e: (e, 0, 0)),
        ],
        out_specs=pl.BlockSpec((CAP, DH), lambda e: (e, 0)),
        out_shape=jax.ShapeDtypeStruct((NPAD, DH), jnp.int32),
        compiler_params=pltpu.CompilerParams(
            dimension_semantics=("arbitrary",)),
    )(buf, fc1_w, fc1_b.reshape(E, 1, DFF), fc2_w, fc2_b.reshape(E, 1, D))


NCH = TPW // CCHUNK


def _gather_body(y_hbm, f0_hbm, f1_hbm, r0_hbm, r1_hbm,
                 idx0_v, idx1_v,
                 rows0_a, rows1_a, rows0_b, rows1_b,
                 sem0a, sem1a, sem0b, sem1b, semw_a, semw_b):
    wid = lax.axis_index("c") * NS + lax.axis_index("s")
    base = wid * TPW
    pltpu.sync_copy(f0_hbm.at[pl.ds(base, TPW)], idx0_v)
    pltpu.sync_copy(f1_hbm.at[pl.ds(base, TPW)], idx1_v)

    bufs = [(rows0_a, rows1_a, sem0a, sem1a), (rows0_b, rows1_b, sem0b, sem1b)]
    wsems = [semw_a, semw_b]

    def gather(c, r0, r1, s0, s1):
        t0 = c * CCHUNK
        h0 = pltpu.async_copy(y_hbm.at[idx0_v.at[pl.ds(t0, CCHUNK)]], r0, s0)
        h1 = pltpu.async_copy(y_hbm.at[idx1_v.at[pl.ds(t0, CCHUNK)]], r1, s1)
        return h0, h1

    handles = [None, None]
    wh = [None, None]
    handles[0] = gather(0, *bufs[0])
    for c in range(NCH):
        cur = c % 2
        nxt = (c + 1) % 2
        if c + 1 < NCH:
            if wh[nxt] is not None:
                wh[nxt][0].wait()
                wh[nxt][1].wait()
            handles[nxt] = gather(c + 1, *bufs[nxt])
        r0, r1, _, _ = bufs[cur]
        handles[cur][0].wait()
        handles[cur][1].wait()
        t0 = base + c * CCHUNK
        w0 = pltpu.async_copy(r0, r0_hbm.at[pl.ds(t0, CCHUNK)], wsems[cur])
        w1 = pltpu.async_copy(r1, r1_hbm.at[pl.ds(t0, CCHUNK)], wsems[cur])
        wh[cur] = (w0, w1)
    for h in wh:
        if h is not None:
            h[0].wait()
            h[1].wait()


def _gather(y, f0, f1):
    mesh = plsc.VectorSubcoreMesh(core_axis_name="c", subcore_axis_name="s")
    return pl.kernel(
        _gather_body,
        out_type=[
            jax.ShapeDtypeStruct((T, DH), jnp.int32),
            jax.ShapeDtypeStruct((T, DH), jnp.int32),
        ],
        mesh=mesh,
        scratch_types=[
            pltpu.VMEM((TPW,), jnp.int32),
            pltpu.VMEM((TPW,), jnp.int32),
            pltpu.VMEM((CCHUNK, DH), jnp.int32),
            pltpu.VMEM((CCHUNK, DH), jnp.int32),
            pltpu.VMEM((CCHUNK, DH), jnp.int32),
            pltpu.VMEM((CCHUNK, DH), jnp.int32),
            pltpu.SemaphoreType.DMA,
            pltpu.SemaphoreType.DMA,
            pltpu.SemaphoreType.DMA,
            pltpu.SemaphoreType.DMA,
            pltpu.SemaphoreType.DMA,
            pltpu.SemaphoreType.DMA,
        ],
    )(y, f0, f1)


MIXB = 256


def _mix_body(r0_ref, r1_ref, g0_ref, g1_ref, out_ref):
    g0 = g0_ref[...]
    g1 = g1_ref[...]
    r0 = _unpack_pairs(r0_ref[...])
    r1 = _unpack_pairs(r1_ref[...])
    out_ref[...] = (jnp.where(g0 > 0, g0 * r0, 0.0)
                    + jnp.where(g1 > 0, g1 * r1, 0.0))


def _mix(r0, r1, g0, g1):
    return pl.pallas_call(
        _mix_body,
        grid=(T // MIXB,),
        in_specs=[
            pl.BlockSpec((MIXB, DH), lambda i: (i, 0)),
            pl.BlockSpec((MIXB, DH), lambda i: (i, 0)),
            pl.BlockSpec((MIXB, 1), lambda i: (i, 0)),
            pl.BlockSpec((MIXB, 1), lambda i: (i, 0)),
        ],
        out_specs=pl.BlockSpec((MIXB, D), lambda i: (i, 0)),
        out_shape=jax.ShapeDtypeStruct((T, D), jnp.float32),
        compiler_params=pltpu.CompilerParams(
            dimension_semantics=("arbitrary",)),
    )(r0, r1, g0, g1)


@jax.jit
def kernel(x, wg, fc1_w, fc1_b, fc2_w, fc2_b):
    f0, f1, g0, g1, xb = _router(x, wg)
    buf = _dispatch(xb, f0, f1)
    y = _ffn(buf, fc1_w, fc1_b, fc2_w, fc2_b)
    r0, r1 = _gather(y, f0, f1)
    return _mix(r0, r1, g0, g1)
